# Initial kernel scaffold; baseline (speedup 1.0000x reference)
import jax
import jax.numpy as jnp
from jax.experimental import pallas as pl


def kernel(edge_index, edge_weight, W_user, W_item):
    raise NotImplementedError("write your pallas kernel here")


# speedup vs baseline: 4.1197x; 4.1197x over previous
import functools

import jax
import jax.numpy as jnp
from jax import lax
from jax.experimental import pallas as pl
from jax.experimental.pallas import tpu as pltpu
from jax.experimental.pallas import tpu_sc as plsc

U_N_ = 60000
I_N_ = 40000
N_ = U_N_ + I_N_
D_ = 32
DH_ = 16
L_LAYERS_ = 3
E_ = 1600000
B_ = 128
NTILE_ = 16
NCORE_ = 2

EPAD_ = ((E_ + B_ * NTILE_ - 1) // (B_ * NTILE_)) * (B_ * NTILE_)
NB_ = EPAD_ // B_
NB_TILE_ = NB_ // NTILE_
NPAD_ = 100096
OROWS_ = NPAD_ // NTILE_
MCH_ = 34
MROWS_ = OROWS_ // MCH_


def _body(emb0_t, src2, dst2, w2, zeros_t,
          t1, t2, t3, out2,
          acc, src_v, dst_v, w_v, rows_v,
          m0, m1, m2, m3, mo):
    c = lax.axis_index("c")
    s = lax.axis_index("s")
    half_off = c * NPAD_

    tables = [emb0_t, t1, t2, t3]

    z0 = s * OROWS_
    pltpu.sync_copy(zeros_t.at[pl.ds(z0, OROWS_)], acc.at[pl.ds(z0, OROWS_)])
    plsc.subcore_barrier()

    def edge_batch(src_tab):
        def body(j):
            gb = s * NB_TILE_ + j
            pltpu.sync_copy(src2.at[gb], src_v)
            pltpu.sync_copy(dst2.at[gb], dst_v)
            pltpu.sync_copy(w2.at[gb], w_v)
            for k in range(B_ // 16):
                sl = pl.ds(k * 16, 16)
                src_v[sl] = src_v[sl] + half_off
            pltpu.sync_copy(src_tab.at[src_v], rows_v)

            def scale(j, _):
                w16 = w_v[pl.ds(j * 16, 16)]
                for k in range(16):
                    i = j * 16 + k
                    rows_v[i] = rows_v[i] * w16[k]
                return 0

            lax.fori_loop(0, B_ // 16, scale, 0)
            pltpu.sync_copy(rows_v, acc.at[dst_v], add=True)
            return 0

        lax.fori_loop(0, NB_TILE_, lambda j, _: body(j), 0)

    r0 = s * OROWS_
    for layer in range(L_LAYERS_):
        edge_batch(tables[layer])
        plsc.subcore_barrier()
        pltpu.sync_copy(acc.at[pl.ds(r0, OROWS_)],
                        tables[layer + 1].at[pl.ds(half_off + r0, OROWS_)])
        plsc.subcore_barrier()
        if layer + 1 < L_LAYERS_:
            pltpu.sync_copy(zeros_t.at[pl.ds(z0, OROWS_)],
                            acc.at[pl.ds(z0, OROWS_)])
            plsc.subcore_barrier()

    for ch in range(MCH_):
        g0 = half_off + r0 + ch * MROWS_
        pltpu.sync_copy(emb0_t.at[pl.ds(g0, MROWS_)], m0)
        pltpu.sync_copy(t1.at[pl.ds(g0, MROWS_)], m1)
        pltpu.sync_copy(t2.at[pl.ds(g0, MROWS_)], m2)
        pltpu.sync_copy(t3.at[pl.ds(g0, MROWS_)], m3)

        def mean_row(i, _):
            mo[i] = (m0[i] + m1[i] + m2[i] + m3[i]) * 0.25
            return 0

        lax.fori_loop(0, MROWS_, mean_row, 0)
        pltpu.sync_copy(mo, out2.at[pl.ds(g0, MROWS_)])


@jax.jit
def _run(emb0_t, src2, dst2, w2, zeros_t):
    mesh = plsc.VectorSubcoreMesh(core_axis_name="c", subcore_axis_name="s")
    f32 = jnp.float32
    out_types = (
        jax.ShapeDtypeStruct((2 * NPAD_, DH_), f32),
        jax.ShapeDtypeStruct((2 * NPAD_, DH_), f32),
        jax.ShapeDtypeStruct((2 * NPAD_, DH_), f32),
        jax.ShapeDtypeStruct((2 * NPAD_, DH_), f32),
    )
    scratch = [
        pltpu.VMEM_SHARED((NPAD_, DH_), f32),
        pltpu.VMEM((B_,), jnp.int32),
        pltpu.VMEM((B_,), jnp.int32),
        pltpu.VMEM((B_,), f32),
        pltpu.VMEM((B_, DH_), f32),
        pltpu.VMEM((MROWS_, DH_), f32),
        pltpu.VMEM((MROWS_, DH_), f32),
        pltpu.VMEM((MROWS_, DH_), f32),
        pltpu.VMEM((MROWS_, DH_), f32),
        pltpu.VMEM((MROWS_, DH_), f32),
    ]
    kfn = pl.kernel(
        _body,
        out_type=out_types,
        scratch_types=scratch,
        mesh=mesh,
        compiler_params=pltpu.CompilerParams(use_tc_tiling_on_sc=False),
    )
    return kfn(emb0_t, src2, dst2, w2, zeros_t)


def kernel(edge_index, edge_weight, W_user, W_item):
    emb0 = jnp.concatenate([W_user, W_item], axis=0)
    rpad = jnp.zeros((NPAD_ - N_, DH_), jnp.float32)
    emb0_t = jnp.concatenate(
        [emb0[:, :DH_], rpad, emb0[:, DH_:], rpad], axis=0)

    src = edge_index[1]
    dst = edge_index[0]
    w = edge_weight
    pad = EPAD_ - E_
    src_p = jnp.concatenate([src, jnp.zeros((pad,), jnp.int32)])
    dst_p = jnp.concatenate([dst, jnp.full((pad,), N_, jnp.int32)])
    w_p = jnp.concatenate([w, jnp.zeros((pad,), jnp.float32)])
    src2 = src_p.reshape(NB_, B_)
    dst2 = dst_p.reshape(NB_, B_)
    w2 = w_p.reshape(NB_, B_)
    zeros_t = jnp.zeros((NPAD_, DH_), jnp.float32)

    _, _, _, out2 = _run(emb0_t, src2, dst2, w2, zeros_t)
    out = jnp.concatenate([out2[:N_], out2[NPAD_:NPAD_ + N_]], axis=1)
    return (emb0, out)


# speedup vs baseline: 9.9664x; 2.4192x over previous
import functools

import jax
import jax.numpy as jnp
from jax import lax
from jax.experimental import pallas as pl
from jax.experimental.pallas import tpu as pltpu
from jax.experimental.pallas import tpu_sc as plsc

U_N_ = 60000
I_N_ = 40000
N_ = U_N_ + I_N_
D_ = 32
DH_ = 16
L_LAYERS_ = 3
E_ = 1600000
B_ = 128
NTILE_ = 16
NCORE_ = 2

EPAD_ = ((E_ + B_ * NTILE_ - 1) // (B_ * NTILE_)) * (B_ * NTILE_)
NB_ = EPAD_ // B_
NB_TILE_ = NB_ // NTILE_
NPAD_ = 100096
OROWS_ = NPAD_ // NTILE_
MCH_ = 46
MROWS_ = OROWS_ // MCH_
CB_ = 34
NCH_ = NB_TILE_ // CB_


def _body(emb0_t, src2, dst2, w2, zeros_t,
          t1, t2, t3, out2,
          acc, src_st, dst_st, w_st,
          rows_a, rows_b, gsa, gsb, ssa, ssb,
          m0, m1, m2, m3, mo):
    c = lax.axis_index("c")
    s = lax.axis_index("s")
    half_off = c * NPAD_

    tables = [emb0_t, t1, t2, t3]

    z0 = s * OROWS_
    pltpu.sync_copy(zeros_t.at[pl.ds(z0, OROWS_)], acc.at[pl.ds(z0, OROWS_)])
    plsc.subcore_barrier()

    def scale(buf, jj):
        for m in range(B_ // 16):
            w16 = w_st[jj, pl.ds(m * 16, 16)]
            for k in range(16):
                i = m * 16 + k
                buf[i] = buf[i] * w16[k]

    def edge_chunks(src_tab):
        def g_start(jj, buf, sem):
            pltpu.async_copy(src_tab.at[src_st.at[jj]], buf, sem)

        def g_wait(buf, sem):
            pltpu.make_async_copy(src_tab.at[src_st.at[0]], buf, sem).wait()

        def s_start(jj, buf, sem):
            pltpu.async_copy(buf, acc.at[dst_st.at[jj]], sem, add=True)

        def s_wait(buf, sem):
            pltpu.make_async_copy(buf, acc.at[dst_st.at[0]], sem).wait()

        def chunk(cb, _):
            base = s * NB_TILE_ + cb * CB_
            pltpu.sync_copy(src2.at[pl.ds(base, CB_)], src_st)
            pltpu.sync_copy(dst2.at[pl.ds(base, CB_)], dst_st)
            pltpu.sync_copy(w2.at[pl.ds(base, CB_)], w_st)

            def add_off(jj, _):
                for k in range(B_ // 16):
                    sl = pl.ds(k * 16, 16)
                    src_st[jj, sl] = src_st[jj, sl] + half_off
                return 0

            lax.fori_loop(0, CB_, add_off, 0)

            g_start(0, rows_a, gsa)

            def pair(t, _):
                g_wait(rows_a, gsa)

                @pl.when(t > 0)
                def _():
                    s_wait(rows_b, ssb)

                g_start(2 * t + 1, rows_b, gsb)
                scale(rows_a, 2 * t)
                s_start(2 * t, rows_a, ssa)
                g_wait(rows_b, gsb)
                s_wait(rows_a, ssa)
                g_start(2 * t + 2, rows_a, gsa)
                scale(rows_b, 2 * t + 1)
                s_start(2 * t + 1, rows_b, ssb)
                return 0

            lax.fori_loop(0, CB_ // 2 - 1, pair, 0)
            g_wait(rows_a, gsa)
            s_wait(rows_b, ssb)
            g_start(CB_ - 1, rows_b, gsb)
            scale(rows_a, CB_ - 2)
            s_start(CB_ - 2, rows_a, ssa)
            g_wait(rows_b, gsb)
            s_wait(rows_a, ssa)
            scale(rows_b, CB_ - 1)
            s_start(CB_ - 1, rows_b, ssb)
            s_wait(rows_b, ssb)
            return 0

        lax.fori_loop(0, NCH_, chunk, 0)

    r0 = s * OROWS_
    for layer in range(L_LAYERS_):
        edge_chunks(tables[layer])
        plsc.subcore_barrier()
        pltpu.sync_copy(acc.at[pl.ds(r0, OROWS_)],
                        tables[layer + 1].at[pl.ds(half_off + r0, OROWS_)])
        plsc.subcore_barrier()
        if layer + 1 < L_LAYERS_:
            pltpu.sync_copy(zeros_t.at[pl.ds(z0, OROWS_)],
                            acc.at[pl.ds(z0, OROWS_)])
            plsc.subcore_barrier()

    for ch in range(MCH_):
        g0 = half_off + r0 + ch * MROWS_
        pltpu.sync_copy(emb0_t.at[pl.ds(g0, MROWS_)], m0)
        pltpu.sync_copy(t1.at[pl.ds(g0, MROWS_)], m1)
        pltpu.sync_copy(t2.at[pl.ds(g0, MROWS_)], m2)
        pltpu.sync_copy(t3.at[pl.ds(g0, MROWS_)], m3)

        def mean_row(i, _):
            mo[i] = (m0[i] + m1[i] + m2[i] + m3[i]) * 0.25
            return 0

        lax.fori_loop(0, MROWS_, mean_row, 0)
        pltpu.sync_copy(mo, out2.at[pl.ds(g0, MROWS_)])


@jax.jit
def _run(emb0_t, src2, dst2, w2, zeros_t):
    mesh = plsc.VectorSubcoreMesh(core_axis_name="c", subcore_axis_name="s")
    f32 = jnp.float32
    out_types = (
        jax.ShapeDtypeStruct((2 * NPAD_, DH_), f32),
        jax.ShapeDtypeStruct((2 * NPAD_, DH_), f32),
        jax.ShapeDtypeStruct((2 * NPAD_, DH_), f32),
        jax.ShapeDtypeStruct((2 * NPAD_, DH_), f32),
    )
    scratch = [
        pltpu.VMEM_SHARED((NPAD_, DH_), f32),
        pltpu.VMEM((CB_, B_), jnp.int32),
        pltpu.VMEM((CB_, B_), jnp.int32),
        pltpu.VMEM((CB_, B_), f32),
        pltpu.VMEM((B_, DH_), f32),
        pltpu.VMEM((B_, DH_), f32),
        pltpu.SemaphoreType.DMA,
        pltpu.SemaphoreType.DMA,
        pltpu.SemaphoreType.DMA,
        pltpu.SemaphoreType.DMA,
        pltpu.VMEM((MROWS_, DH_), f32),
        pltpu.VMEM((MROWS_, DH_), f32),
        pltpu.VMEM((MROWS_, DH_), f32),
        pltpu.VMEM((MROWS_, DH_), f32),
        pltpu.VMEM((MROWS_, DH_), f32),
    ]
    kfn = pl.kernel(
        _body,
        out_type=out_types,
        scratch_types=scratch,
        mesh=mesh,
        compiler_params=pltpu.CompilerParams(use_tc_tiling_on_sc=False),
    )
    return kfn(emb0_t, src2, dst2, w2, zeros_t)


def kernel(edge_index, edge_weight, W_user, W_item):
    emb0 = jnp.concatenate([W_user, W_item], axis=0)
    rpad = jnp.zeros((NPAD_ - N_, DH_), jnp.float32)
    emb0_t = jnp.concatenate(
        [emb0[:, :DH_], rpad, emb0[:, DH_:], rpad], axis=0)

    src = edge_index[1]
    dst = edge_index[0]
    w = edge_weight
    pad = EPAD_ - E_
    src_p = jnp.concatenate([src, jnp.zeros((pad,), jnp.int32)])
    dst_p = jnp.concatenate([dst, jnp.full((pad,), N_, jnp.int32)])
    w_p = jnp.concatenate([w, jnp.zeros((pad,), jnp.float32)])
    src2 = src_p.reshape(NB_, B_)
    dst2 = dst_p.reshape(NB_, B_)
    w2 = w_p.reshape(NB_, B_)
    zeros_t = jnp.zeros((NPAD_, DH_), jnp.float32)

    _, _, _, out2 = _run(emb0_t, src2, dst2, w2, zeros_t)
    out = jnp.concatenate([out2[:N_], out2[NPAD_:NPAD_ + N_]], axis=1)
    return (emb0, out)


# speedup vs baseline: 10.2434x; 1.0278x over previous
import jax
import jax.numpy as jnp
from jax import lax
from jax.experimental import pallas as pl
from jax.experimental.pallas import tpu as pltpu
from jax.experimental.pallas import tpu_sc as plsc

U_N_ = 60000
I_N_ = 40000
N_ = U_N_ + I_N_
D_ = 32
DH_ = 16
L_LAYERS_ = 3
E_ = 1600000
B_ = 128
NTILE_ = 16
G_ = 4

NB_TILE_ = 792
NB_ = NB_TILE_ * NTILE_
EPAD_ = NB_ * B_
NPAD_ = 100096
OROWS_ = NPAD_ // NTILE_
CB_ = 24
NCH_ = NB_TILE_ // CB_
MFULL_ = OROWS_ // B_
MREM_ = OROWS_ - MFULL_ * B_


def _body(emb0_t, src2, dst2, w2, zeros_t,
          tbl, out2,
          acc, src_st, dst_st, w_st,
          ra0, ra1, ra2, ra3, rb0, rb1, rb2, rb3,
          gsa, gsb, ssa, ssb):
    c = lax.axis_index("c")
    s = lax.axis_index("s")
    half_off = c * NPAD_

    grp_a = [ra0, ra1, ra2, ra3]
    grp_b = [rb0, rb1, rb2, rb3]

    z0 = s * OROWS_
    r0 = s * OROWS_
    pltpu.sync_copy(zeros_t.at[pl.ds(z0, OROWS_)], acc.at[pl.ds(z0, OROWS_)])
    pltpu.sync_copy(emb0_t.at[pl.ds(half_off + r0, OROWS_)],
                    tbl.at[0, pl.ds(half_off + r0, OROWS_)])
    plsc.subcore_barrier()

    def scale(buf, jj):
        for m in range(B_ // 16):
            w16 = w_st[jj, pl.ds(m * 16, 16)]
            for k in range(16):
                i = m * 16 + k
                buf[i] = buf[i] * w16[k]

    def edge_chunks(src_tab):
        def g_start(jg, bufs, sem):
            for b in range(G_):
                pltpu.async_copy(src_tab.at[src_st.at[jg + b]], bufs[b], sem)

        def g_wait(bufs, sem):
            for b in range(G_):
                pltpu.make_async_copy(src_tab.at[src_st.at[0]], bufs[b],
                                      sem).wait()

        def s_start(jg, bufs, sem):
            for b in range(G_):
                pltpu.async_copy(bufs[b], acc.at[dst_st.at[jg + b]], sem,
                                 add=True)

        def s_wait(bufs, sem):
            for b in range(G_):
                pltpu.make_async_copy(bufs[b], acc.at[dst_st.at[0]],
                                      sem).wait()

        def scale4(jg, bufs):
            for b in range(G_):
                scale(bufs[b], jg + b)

        def chunk(cb, _):
            base = s * NB_TILE_ + cb * CB_
            pltpu.sync_copy(src2.at[pl.ds(base, CB_)], src_st)
            pltpu.sync_copy(dst2.at[pl.ds(base, CB_)], dst_st)
            pltpu.sync_copy(w2.at[pl.ds(base, CB_)], w_st)

            def add_off(jj, _):
                for k in range(B_ // 16):
                    sl = pl.ds(k * 16, 16)
                    src_st[jj, sl] = src_st[jj, sl] + half_off
                return 0

            lax.fori_loop(0, CB_, add_off, 0)

            nsp = CB_ // (2 * G_)
            g_start(0, grp_a, gsa)

            def superpair(t, _):
                jA = 2 * G_ * t
                jB = jA + G_
                g_wait(grp_a, gsa)

                @pl.when(t > 0)
                def _():
                    s_wait(grp_b, ssb)

                g_start(jB, grp_b, gsb)
                scale4(jA, grp_a)
                s_start(jA, grp_a, ssa)
                g_wait(grp_b, gsb)
                s_wait(grp_a, ssa)

                @pl.when(t < nsp - 1)
                def _():
                    g_start(jA + 2 * G_, grp_a, gsa)

                scale4(jB, grp_b)
                s_start(jB, grp_b, ssb)
                return 0

            lax.fori_loop(0, nsp, superpair, 0)
            s_wait(grp_b, ssb)
            return 0

        lax.fori_loop(0, NCH_, chunk, 0)

    def layer_step(layer, _):
        edge_chunks(tbl.at[layer])
        plsc.subcore_barrier()
        pltpu.sync_copy(acc.at[pl.ds(r0, OROWS_)],
                        tbl.at[layer + 1, pl.ds(half_off + r0, OROWS_)])
        plsc.subcore_barrier()
        pltpu.sync_copy(zeros_t.at[pl.ds(z0, OROWS_)],
                        acc.at[pl.ds(z0, OROWS_)])
        plsc.subcore_barrier()
        return 0

    lax.fori_loop(0, L_LAYERS_, layer_step, 0)

    def mean_rows(nrows):
        def body(i, _):
            rb0[i] = (ra0[i] + ra1[i] + ra2[i] + ra3[i]) * 0.25
            return 0

        lax.fori_loop(0, nrows, body, 0)

    def mean_chunk(ch, _):
        g0 = half_off + r0 + ch * B_
        pltpu.sync_copy(tbl.at[0, pl.ds(g0, B_)], ra0)
        pltpu.sync_copy(tbl.at[1, pl.ds(g0, B_)], ra1)
        pltpu.sync_copy(tbl.at[2, pl.ds(g0, B_)], ra2)
        pltpu.sync_copy(tbl.at[3, pl.ds(g0, B_)], ra3)
        mean_rows(B_)
        pltpu.sync_copy(rb0, out2.at[pl.ds(g0, B_)])
        return 0

    lax.fori_loop(0, MFULL_, mean_chunk, 0)
    g0 = half_off + r0 + MFULL_ * B_
    pltpu.sync_copy(tbl.at[0, pl.ds(g0, MREM_)], ra0.at[pl.ds(0, MREM_)])
    pltpu.sync_copy(tbl.at[1, pl.ds(g0, MREM_)], ra1.at[pl.ds(0, MREM_)])
    pltpu.sync_copy(tbl.at[2, pl.ds(g0, MREM_)], ra2.at[pl.ds(0, MREM_)])
    pltpu.sync_copy(tbl.at[3, pl.ds(g0, MREM_)], ra3.at[pl.ds(0, MREM_)])
    mean_rows(MREM_)
    pltpu.sync_copy(rb0.at[pl.ds(0, MREM_)], out2.at[pl.ds(g0, MREM_)])


@jax.jit
def _run(emb0_t, src2, dst2, w2, zeros_t):
    mesh = plsc.VectorSubcoreMesh(core_axis_name="c", subcore_axis_name="s")
    f32 = jnp.float32
    out_types = (
        jax.ShapeDtypeStruct((4, 2 * NPAD_, DH_), f32),
        jax.ShapeDtypeStruct((2 * NPAD_, DH_), f32),
    )
    scratch = [
        pltpu.VMEM_SHARED((NPAD_, DH_), f32),
        pltpu.VMEM((CB_, B_), jnp.int32),
        pltpu.VMEM((CB_, B_), jnp.int32),
        pltpu.VMEM((CB_, B_), f32),
    ]
    scratch += [pltpu.VMEM((B_, DH_), f32) for _ in range(8)]
    scratch += [pltpu.SemaphoreType.DMA] * 4
    kfn = pl.kernel(
        _body,
        out_type=out_types,
        scratch_types=scratch,
        mesh=mesh,
        compiler_params=pltpu.CompilerParams(use_tc_tiling_on_sc=False),
    )
    return kfn(emb0_t, src2, dst2, w2, zeros_t)


def kernel(edge_index, edge_weight, W_user, W_item):
    emb0 = jnp.concatenate([W_user, W_item], axis=0)
    rpad = jnp.zeros((NPAD_ - N_, DH_), jnp.float32)
    emb0_t = jnp.concatenate(
        [emb0[:, :DH_], rpad, emb0[:, DH_:], rpad], axis=0)

    src = edge_index[1]
    dst = edge_index[0]
    w = edge_weight
    pad = EPAD_ - E_
    src_p = jnp.concatenate([src, jnp.zeros((pad,), jnp.int32)])
    dst_p = jnp.concatenate([dst, jnp.full((pad,), N_, jnp.int32)])
    w_p = jnp.concatenate([w, jnp.zeros((pad,), jnp.float32)])
    src2 = src_p.reshape(NB_, B_)
    dst2 = dst_p.reshape(NB_, B_)
    w2 = w_p.reshape(NB_, B_)
    zeros_t = jnp.zeros((NPAD_, DH_), jnp.float32)

    _, out2 = _run(emb0_t, src2, dst2, w2, zeros_t)
    out = jnp.concatenate([out2[:N_], out2[NPAD_:NPAD_ + N_]], axis=1)
    return (emb0, out)


# speedup vs baseline: 12.4957x; 1.2199x over previous
import jax
import jax.numpy as jnp
from jax import lax
from jax.experimental import pallas as pl
from jax.experimental.pallas import tpu as pltpu
from jax.experimental.pallas import tpu_sc as plsc

U_N_ = 60000
I_N_ = 40000
N_ = U_N_ + I_N_
D_ = 32
DH_ = 16
L_LAYERS_ = 3
E_ = 1600000
B_ = 128
NTILE_ = 16
G_ = 4

NB_TILE_ = 792
NB_ = NB_TILE_ * NTILE_
EPAD_ = NB_ * B_
NPAD_ = 100096
OROWS_ = NPAD_ // NTILE_
CB_ = 24
NCH_ = NB_TILE_ // CB_
MFULL_ = OROWS_ // B_
MREM_ = OROWS_ - MFULL_ * B_


def _body(emb0_t, src2, dst2, w2,
          tbl, out2,
          acc, src_st, dst_st, w_st,
          ra0, ra1, ra2, ra3, rb0, rb1, rb2, rb3,
          gsa, gsb, ssa, ssb):
    c = lax.axis_index("c")
    s = lax.axis_index("s")
    half_off = c * NPAD_

    grp_a = [ra0, ra1, ra2, ra3]
    grp_b = [rb0, rb1, rb2, rb3]
    r0 = s * OROWS_

    def zero_buf(buf):
        zv = jnp.zeros((16,), jnp.float32)

        def zrow(i, _):
            buf[i] = zv
            return 0

        lax.fori_loop(0, B_, zrow, 0)

    def zero_acc_slice():
        zero_buf(rb3)

        def zchunk(ch, _):
            pltpu.sync_copy(rb3, acc.at[pl.ds(r0 + ch * B_, B_)])
            return 0

        lax.fori_loop(0, MFULL_, zchunk, 0)
        pltpu.sync_copy(rb3.at[pl.ds(0, MREM_)],
                        acc.at[pl.ds(r0 + MFULL_ * B_, MREM_)])

    zero_acc_slice()
    plsc.subcore_barrier()

    def scale(buf, jj):
        for m in range(B_ // 16):
            w16 = w_st[jj, pl.ds(m * 16, 16)]
            for k in range(16):
                i = m * 16 + k
                buf[i] = buf[i] * w16[k]

    def edge_chunks(src_tab):
        def g_start(jg, bufs, sem):
            for b in range(G_):
                pltpu.async_copy(src_tab.at[src_st.at[jg + b]], bufs[b], sem)

        def g_wait(bufs, sem):
            for b in range(G_):
                pltpu.make_async_copy(src_tab.at[src_st.at[0]], bufs[b],
                                      sem).wait()

        def s_start(jg, bufs, sem):
            for b in range(G_):
                pltpu.async_copy(bufs[b], acc.at[dst_st.at[jg + b]], sem,
                                 add=True)

        def s_wait(bufs, sem):
            for b in range(G_):
                pltpu.make_async_copy(bufs[b], acc.at[dst_st.at[0]],
                                      sem).wait()

        def scale4(jg, bufs):
            for b in range(G_):
                scale(bufs[b], jg + b)

        def chunk(cb, _):
            base = s * NB_TILE_ + cb * CB_
            pltpu.sync_copy(src2.at[pl.ds(base, CB_)], src_st)
            pltpu.sync_copy(dst2.at[pl.ds(base, CB_)], dst_st)
            pltpu.sync_copy(w2.at[pl.ds(base, CB_)], w_st)

            def add_off(jj, _):
                for k in range(B_ // 16):
                    sl = pl.ds(k * 16, 16)
                    src_st[jj, sl] = src_st[jj, sl] + half_off
                return 0

            lax.fori_loop(0, CB_, add_off, 0)

            nsp = CB_ // (2 * G_)
            g_start(0, grp_a, gsa)

            def superpair(t, _):
                jA = 2 * G_ * t
                jB = jA + G_
                g_wait(grp_a, gsa)

                @pl.when(t > 0)
                def _():
                    s_wait(grp_b, ssb)

                g_start(jB, grp_b, gsb)
                scale4(jA, grp_a)
                s_start(jA, grp_a, ssa)
                g_wait(grp_b, gsb)
                s_wait(grp_a, ssa)

                @pl.when(t < nsp - 1)
                def _():
                    g_start(jA + 2 * G_, grp_a, gsa)

                scale4(jB, grp_b)
                s_start(jB, grp_b, ssb)
                return 0

            lax.fori_loop(0, nsp, superpair, 0)
            s_wait(grp_b, ssb)
            return 0

        lax.fori_loop(0, NCH_, chunk, 0)

    def publish_and_rezero(layer):
        plsc.subcore_barrier()
        pltpu.sync_copy(acc.at[pl.ds(r0, OROWS_)],
                        tbl.at[layer, pl.ds(half_off + r0, OROWS_)])
        zero_acc_slice()
        plsc.subcore_barrier()

    edge_chunks(emb0_t)
    publish_and_rezero(0)

    def layer_step(lay, _):
        edge_chunks(tbl.at[lay])
        publish_and_rezero(lay + 1)
        return 0

    lax.fori_loop(0, L_LAYERS_ - 1, layer_step, 0)

    def mean_rows(nrows):
        def body(i, _):
            rb0[i] = (ra0[i] + ra1[i] + ra2[i] + ra3[i]) * 0.25
            return 0

        lax.fori_loop(0, nrows, body, 0)

    def mean_chunk(ch, _):
        g0 = half_off + r0 + ch * B_
        pltpu.sync_copy(emb0_t.at[pl.ds(g0, B_)], ra0)
        pltpu.sync_copy(tbl.at[0, pl.ds(g0, B_)], ra1)
        pltpu.sync_copy(tbl.at[1, pl.ds(g0, B_)], ra2)
        pltpu.sync_copy(tbl.at[2, pl.ds(g0, B_)], ra3)
        mean_rows(B_)
        pltpu.sync_copy(rb0, out2.at[pl.ds(g0, B_)])
        return 0

    lax.fori_loop(0, MFULL_, mean_chunk, 0)
    g0 = half_off + r0 + MFULL_ * B_
    pltpu.sync_copy(emb0_t.at[pl.ds(g0, MREM_)], ra0.at[pl.ds(0, MREM_)])
    pltpu.sync_copy(tbl.at[0, pl.ds(g0, MREM_)], ra1.at[pl.ds(0, MREM_)])
    pltpu.sync_copy(tbl.at[1, pl.ds(g0, MREM_)], ra2.at[pl.ds(0, MREM_)])
    pltpu.sync_copy(tbl.at[2, pl.ds(g0, MREM_)], ra3.at[pl.ds(0, MREM_)])
    mean_rows(MREM_)
    pltpu.sync_copy(rb0.at[pl.ds(0, MREM_)], out2.at[pl.ds(g0, MREM_)])


@jax.jit
def _run(emb0_t, src2, dst2, w2):
    mesh = plsc.VectorSubcoreMesh(core_axis_name="c", subcore_axis_name="s")
    f32 = jnp.float32
    out_types = (
        jax.ShapeDtypeStruct((3, 2 * NPAD_, DH_), f32),
        jax.ShapeDtypeStruct((2 * NPAD_, DH_), f32),
    )
    scratch = [
        pltpu.VMEM_SHARED((NPAD_, DH_), f32),
        pltpu.VMEM((CB_, B_), jnp.int32),
        pltpu.VMEM((CB_, B_), jnp.int32),
        pltpu.VMEM((CB_, B_), f32),
    ]
    scratch += [pltpu.VMEM((B_, DH_), f32) for _ in range(8)]
    scratch += [pltpu.SemaphoreType.DMA] * 4
    kfn = pl.kernel(
        _body,
        out_type=out_types,
        scratch_types=scratch,
        mesh=mesh,
        compiler_params=pltpu.CompilerParams(use_tc_tiling_on_sc=False),
    )
    return kfn(emb0_t, src2, dst2, w2)


def kernel(edge_index, edge_weight, W_user, W_item):
    emb0 = jnp.concatenate([W_user, W_item], axis=0)
    rpad = jnp.zeros((NPAD_ - N_, DH_), jnp.float32)
    emb0_t = jnp.concatenate(
        [emb0[:, :DH_], rpad, emb0[:, DH_:], rpad], axis=0)

    src = edge_index[1]
    dst = edge_index[0]
    w = edge_weight
    pad = EPAD_ - E_
    src_p = jnp.concatenate([src, jnp.zeros((pad,), jnp.int32)])
    dst_p = jnp.concatenate([dst, jnp.full((pad,), N_, jnp.int32)])
    w_p = jnp.concatenate([w, jnp.zeros((pad,), jnp.float32)])
    src2 = src_p.reshape(NB_, B_)
    dst2 = dst_p.reshape(NB_, B_)
    w2 = w_p.reshape(NB_, B_)

    _, out2 = _run(emb0_t, src2, dst2, w2)
    out = jnp.concatenate([out2[:N_], out2[NPAD_:NPAD_ + N_]], axis=1)
    return (emb0, out)
